# baseline (speedup 1.0000x reference)
import functools

import jax
import jax.numpy as jnp
from jax import lax
from jax.experimental import pallas as pl
from jax.experimental.pallas import tpu as pltpu
from jax.experimental.pallas import tpu_sc as plsc

NUM_EXPERTS = 64
K = 8
NUM_SHARED = 2
K_SELECT = K - NUM_SHARED
END_IDX = NUM_EXPERTS - NUM_SHARED

BATCH = 32768
NC = 2
NS = 16
L = 16
NW = NC * NS
ROWS_PER_W = BATCH // NW
CHUNK_ROWS = 256
CHUNKS = ROWS_PER_W // CHUNK_ROWS
GROUPS_PER_CHUNK = CHUNK_ROWS // L

_mesh = plsc.VectorSubcoreMesh(
    core_axis_name="c", subcore_axis_name="s", num_cores=NC, num_subcores=NS)


@functools.partial(
    pl.kernel,
    out_type=[
        jax.ShapeDtypeStruct((BATCH * K,), jnp.int32),
        jax.ShapeDtypeStruct((BATCH * K,), jnp.float32),
    ],
    mesh=_mesh,
    scratch_types=[
        pltpu.VMEM((CHUNK_ROWS, NUM_EXPERTS), jnp.float32),
        pltpu.VMEM((CHUNK_ROWS * K,), jnp.int32),
        pltpu.VMEM((CHUNK_ROWS * K,), jnp.float32),
    ],
    compiler_params=pltpu.CompilerParams(needs_layout_passes=False),
)
def _sc_topk(gates_hbm, idx_hbm, gate_hbm, buf, oidx, ogate):
    wid = lax.axis_index("s") * NC + lax.axis_index("c")
    row_base = wid * ROWS_PER_W

    lane = lax.iota(jnp.int32, L)
    lane_out = lane * K
    min_key = jnp.full((L,), jnp.iinfo(jnp.int32).min, dtype=jnp.int32)
    i62 = jnp.full((L,), END_IDX, dtype=jnp.int32)
    i63 = jnp.full((L,), END_IDX + 1, dtype=jnp.int32)

    def group_body(g, carry):
        row = g * L + lane
        obase = g * (L * K) + lane_out

        t = [min_key] * K_SELECT
        for j in range(END_IDX):
            cj = jnp.full((L,), j, dtype=jnp.int32)
            v = plsc.load_gather(buf, [row, cj])
            b = plsc.bitcast(v, jnp.int32)
            m = b ^ (jnp.right_shift(b, 31) & jnp.int32(0x7FFFFFFF))
            key = (m & jnp.int32(~63)) | jnp.int32(63 - j)
            for lvl in range(K_SELECT):
                hi = jnp.maximum(key, t[lvl])
                if lvl < K_SELECT - 1:
                    key = jnp.minimum(key, t[lvl])
                t[lvl] = hi

        idxs = [jnp.int32(63) - (t[lvl] & jnp.int32(63))
                for lvl in range(K_SELECT)]
        vals = [plsc.load_gather(buf, [row, idxs[lvl]])
                for lvl in range(K_SELECT)]
        s62 = plsc.load_gather(buf, [row, i62])
        s63 = plsc.load_gather(buf, [row, i63])

        vals = vals + [s62, s63]
        idxs = idxs + [i62, i63]

        mx = jnp.maximum(jnp.maximum(vals[0], s62), s63)
        es = [jnp.exp(x - mx) for x in vals]
        total = ((es[0] + es[1]) + (es[2] + es[3])) + (
            (es[4] + es[5]) + (es[6] + es[7]))
        r = 1.0 / total

        for k in range(K):
            pos = obase + k
            plsc.store_scatter(oidx, [pos], idxs[k])
            plsc.store_scatter(ogate, [pos], es[k] * r)
        return carry

    def chunk_body(ch, carry):
        crow = row_base + ch * CHUNK_ROWS
        pltpu.sync_copy(gates_hbm.at[pl.ds(crow, CHUNK_ROWS), :], buf)
        lax.fori_loop(0, GROUPS_PER_CHUNK, group_body, 0)
        obase = (row_base + ch * CHUNK_ROWS) * K
        pltpu.sync_copy(oidx, idx_hbm.at[pl.ds(obase, CHUNK_ROWS * K)])
        pltpu.sync_copy(ogate, gate_hbm.at[pl.ds(obase, CHUNK_ROWS * K)])
        return carry

    lax.fori_loop(0, CHUNKS, chunk_body, 0)


@jax.jit
def kernel(gates):
    batch = gates.shape[0]
    idx_flat, gate_flat = _sc_topk(gates)
    return idx_flat.reshape(batch, K), gate_flat.reshape(batch, K)


# speedup vs baseline: 2.9619x; 2.9619x over previous
import functools

import jax
import jax.numpy as jnp
from jax import lax
from jax.experimental import pallas as pl
from jax.experimental.pallas import tpu as pltpu
from jax.experimental.pallas import tpu_sc as plsc

NUM_EXPERTS = 64
K = 8
NUM_SHARED = 2
K_SELECT = K - NUM_SHARED
END_IDX = NUM_EXPERTS - NUM_SHARED

BATCH = 32768
NC = 2
NS = 16
L = 16
NW = NC * NS
ROWS_PER_W = BATCH // NW
NSTREAM = 1
CHUNK_ROWS = 1024
CHUNKS = ROWS_PER_W // CHUNK_ROWS
GROUPS_PER_CHUNK = CHUNK_ROWS // L

_mesh = plsc.VectorSubcoreMesh(
    core_axis_name="c", subcore_axis_name="s", num_cores=NC, num_subcores=NS)


@functools.partial(
    pl.kernel,
    out_type=[
        jax.ShapeDtypeStruct((K, BATCH), jnp.int32),
        jax.ShapeDtypeStruct((K, BATCH), jnp.float32),
    ],
    mesh=_mesh,
    scratch_types=[
        pltpu.VMEM((NUM_EXPERTS, CHUNK_ROWS), jnp.float32),
        pltpu.VMEM((K, CHUNK_ROWS), jnp.int32),
        pltpu.VMEM((K, CHUNK_ROWS), jnp.float32),
    ],
    compiler_params=pltpu.CompilerParams(needs_layout_passes=False),
)
def _sc_topk(gates_t_hbm, idx_hbm, gate_hbm, buf, oidx, ogate):
    wid = lax.axis_index("s") * NC + lax.axis_index("c")
    row_base = wid * ROWS_PER_W

    lane = lax.iota(jnp.int32, L)
    min_key = jnp.full((L,), jnp.iinfo(jnp.int32).min, dtype=jnp.int32)
    i62 = jnp.full((L,), END_IDX, dtype=jnp.int32)
    i63 = jnp.full((L,), END_IDX + 1, dtype=jnp.int32)

    def group_body(ch, g, carry):
        cols = [(g * NSTREAM + u) * L for u in range(NSTREAM)]

        t = [[min_key] * K_SELECT for _ in range(NSTREAM)]
        for j in range(END_IDX):
            for u in range(NSTREAM):
                v = buf[j, pl.ds(cols[u], L)]
                b = plsc.bitcast(v, jnp.int32)
                m = b ^ (jnp.right_shift(b, 31) & jnp.int32(0x7FFFFFFF))
                key = (m & jnp.int32(~63)) | jnp.int32(63 - j)
                for lvl in range(K_SELECT):
                    hi = jnp.maximum(key, t[u][lvl])
                    if lvl < K_SELECT - 1:
                        key = jnp.minimum(key, t[u][lvl])
                    t[u][lvl] = hi

        for u in range(NSTREAM):
            colv = jnp.int32(cols[u]) + lane
            idxs = [jnp.int32(63) - (t[u][lvl] & jnp.int32(63))
                    for lvl in range(K_SELECT)]
            vals = [plsc.load_gather(buf, [idxs[lvl], colv])
                    for lvl in range(K_SELECT)]
            s62 = buf[END_IDX, pl.ds(cols[u], L)]
            s63 = buf[END_IDX + 1, pl.ds(cols[u], L)]

            vals = vals + [s62, s63]
            idxs = idxs + [i62, i63]

            mx = jnp.maximum(jnp.maximum(vals[0], s62), s63)
            es = [jnp.exp(x - mx) for x in vals]
            total = ((es[0] + es[1]) + (es[2] + es[3])) + (
                (es[4] + es[5]) + (es[6] + es[7]))
            r = 1.0 / total

            for k in range(K):
                oidx[k, pl.ds(cols[u], L)] = idxs[k]
                ogate[k, pl.ds(cols[u], L)] = es[k] * r
        return carry

    def chunk_body(ch, carry):
        crow = row_base + ch * CHUNK_ROWS
        pltpu.sync_copy(gates_t_hbm.at[:, pl.ds(crow, CHUNK_ROWS)], buf)
        lax.fori_loop(0, GROUPS_PER_CHUNK // NSTREAM,
                      functools.partial(group_body, ch), 0)
        pltpu.sync_copy(oidx, idx_hbm.at[:, pl.ds(crow, CHUNK_ROWS)])
        pltpu.sync_copy(ogate, gate_hbm.at[:, pl.ds(crow, CHUNK_ROWS)])
        return carry

    lax.fori_loop(0, CHUNKS, chunk_body, 0)


@jax.jit
def kernel(gates):
    batch = gates.shape[0]
    out_idx_t, out_gate_t = _sc_topk(gates.T)
    return out_idx_t.T, out_gate_t.T
